# baseline (speedup 1.0000x reference)
import functools

import jax
import jax.numpy as jnp
from jax.experimental import pallas as pl
from jax.experimental.pallas import tpu as pltpu

LN_EPS = 1e-5
V, L, TAU = 32, 8, 1.0


def _ln(x, g, b):
    m = x.mean(-1, keepdims=True)
    v = jnp.mean((x - m) ** 2, axis=-1, keepdims=True)
    return (x - m) * jax.lax.rsqrt(v + LN_EPS) * g + b


def _gelu(x):
    return 0.5 * x * (1.0 + jax.lax.erf(x * 0.7071067811865476))


def _dotT(x, w):
    return jax.lax.dot_general(
        x, w, (((1,), (1,)), ((), ())),
        preferred_element_type=jnp.float32)


def _partial_sum_kernel(h_ref, out_ref, acc, ring, sems,
                        *, cs, depth, n_blocks, n_chunks):
    i = pl.program_id(0)
    k = pl.program_id(1)
    g = i * n_blocks + k

    def in_cp(blk, slot):
        return pltpu.make_async_copy(
            h_ref.at[pl.ds(blk * cs, cs), :], ring.at[slot], sems.at[slot])

    @pl.when(k == 0)
    def _():
        acc[...] = jnp.zeros_like(acc)
        for d in range(depth):
            in_cp(g + d, d).start()

    slot = jax.lax.rem(k, depth)
    in_cp(g, slot).wait()
    b = g // n_chunks
    s = jnp.sum(ring[pl.ds(slot, 1)], axis=1)
    acc[pl.ds(b, 1)] += s[:, None, :]

    @pl.when(k + depth < n_blocks)
    def _():
        in_cp(g + depth, slot).start()

    @pl.when(k == n_blocks - 1)
    def _():
        out_ref[0] = acc[:, 0, :]


def _decode_mlp(parts_ref, gum_ref, w1_ref, b1_ref, g1_ref, be1_ref,
                w2_ref, b2_ref, emb_ref, dw1_ref, db1_ref, g2_ref, be2_ref,
                dw2_ref, db2_ref, n_seq):
    pooled = (parts_ref[0] + parts_ref[1]) * (1.0 / n_seq)
    h = _ln(_dotT(pooled, w1_ref[...]) + b1_ref[...], g1_ref[...], be1_ref[...])
    h = _gelu(h)

    g = -jnp.log(-jnp.log(gum_ref[...]))
    z = _dotT(h, w2_ref[...]) + b2_ref[...] + g

    embed = emb_ref[...]
    acc = jnp.zeros((h.shape[0], dw1_ref.shape[0]), jnp.float32)
    e_dim = emb_ref.shape[1]
    for l in range(L):
        z_l = z[:, l * V:(l + 1) * V]
        m = jnp.max(z_l, axis=-1, keepdims=True)
        iota = jax.lax.broadcasted_iota(jnp.int32, z_l.shape, 1)
        idx = jnp.min(jnp.where(z_l == m, iota, V), axis=-1, keepdims=True)
        y_hard = (iota == idx).astype(jnp.float32)
        emb_l = jnp.dot(y_hard, embed, preferred_element_type=jnp.float32)
        dw1_l = dw1_ref[:, l * e_dim:(l + 1) * e_dim]
        acc = acc + _dotT(emb_l, dw1_l)
    dh = _ln(acc + db1_ref[...], g2_ref[...], be2_ref[...])
    dh = _gelu(dh)
    return _dotT(dh, dw2_ref[...]) + db2_ref[...]


def _add_mlp_kernel(h_ref, parts_ref, gum_ref, w1_ref, b1_ref, g1_ref,
                    be1_ref, w2_ref, b2_ref, emb_ref, dw1_ref, db1_ref,
                    g2_ref, be2_ref, dw2_ref, db2_ref, out_ref, dec_ref,
                    in_ring, out_ring, in_sems, out_sems,
                    *, n_seq, ca, depth, n_blocks, n_chunks):
    i = pl.program_id(0)
    k = pl.program_id(1)
    g = i * n_blocks + k

    def in_cp(blk, slot):
        return pltpu.make_async_copy(
            h_ref.at[pl.ds(blk * ca, ca), :], in_ring.at[slot],
            in_sems.at[slot])

    def out_cp(blk, slot):
        return pltpu.make_async_copy(
            out_ring.at[slot], out_ref.at[pl.ds(blk * ca, ca), :],
            out_sems.at[slot])

    @pl.when(k == 0)
    def _():
        for d in range(depth):
            in_cp(g + d, d).start()
        decoded = _decode_mlp(
            parts_ref, gum_ref, w1_ref, b1_ref, g1_ref, be1_ref, w2_ref,
            b2_ref, emb_ref, dw1_ref, db1_ref, g2_ref, be2_ref, dw2_ref,
            db2_ref, n_seq)
        dec_ref[...] = decoded[:, None, :] * 0.1

    slot = jax.lax.rem(k, depth)
    in_cp(g, slot).wait()

    @pl.when(k >= depth)
    def _():
        out_cp(g - depth, slot).wait()

    b = g // n_chunks
    out_ring[pl.ds(slot, 1)] = (in_ring[pl.ds(slot, 1)]
                                + dec_ref[pl.ds(b, 1), :, :])
    out_cp(g, slot).start()

    @pl.when(k + depth < n_blocks)
    def _():
        in_cp(g + depth, slot).start()

    @pl.when(k == n_blocks - 1)
    def _():
        for d in range(depth):
            blk = n_blocks - depth + d
            out_cp(i * n_blocks + blk, blk % depth).wait()


@jax.jit
def kernel(hidden_states, gumbel_u, enc_w1, enc_b1, ln1_g, ln1_b, enc_w2,
           enc_b2, embed, dec_w1, dec_b1, ln2_g, ln2_b, dec_w2, dec_b2):
    B, S, H = hidden_states.shape
    f32 = jnp.float32
    hidden_flat = hidden_states.reshape(B * S, H)

    CS = 1024
    RDEPTH = 4
    RNBLK = (B * S) // CS // 2
    parts = pl.pallas_call(
        functools.partial(_partial_sum_kernel, cs=CS, depth=RDEPTH,
                          n_blocks=RNBLK, n_chunks=S // CS),
        grid=(2, RNBLK),
        in_specs=[pl.BlockSpec(memory_space=pl.ANY)],
        out_specs=pl.BlockSpec((1, B, H), lambda i, j: (i, 0, 0)),
        out_shape=jax.ShapeDtypeStruct((2, B, H), f32),
        scratch_shapes=[pltpu.VMEM((B, 1, H), f32),
                        pltpu.VMEM((RDEPTH, CS, H), f32),
                        pltpu.SemaphoreType.DMA((RDEPTH,))],
        compiler_params=pltpu.CompilerParams(
            dimension_semantics=("parallel", "arbitrary"),
            vmem_limit_bytes=64 * 1024 * 1024),
    )(hidden_flat)

    CA = 1024
    DEPTH = 3
    NBLK = (B * S) // CA // 2
    NA = S // CA
    vec = lambda v: v.reshape(1, -1)
    whole = lambda a: pl.BlockSpec(a.shape, lambda i, k: (0,) * a.ndim)
    smalls = (parts, gumbel_u.reshape(B, -1), enc_w1, vec(enc_b1), vec(ln1_g), vec(ln1_b),
              enc_w2, vec(enc_b2), embed, dec_w1, vec(dec_b1), vec(ln2_g),
              vec(ln2_b), dec_w2, vec(dec_b2))
    out = pl.pallas_call(
        functools.partial(_add_mlp_kernel, n_seq=S, ca=CA, depth=DEPTH,
                          n_blocks=NBLK, n_chunks=NA),
        grid=(2, NBLK),
        in_specs=[pl.BlockSpec(memory_space=pl.ANY)]
                 + [whole(a) for a in smalls],
        out_specs=pl.BlockSpec(memory_space=pl.ANY),
        out_shape=jax.ShapeDtypeStruct((B * S, H), f32),
        scratch_shapes=[pltpu.VMEM((B, 1, H), f32),
                        pltpu.VMEM((DEPTH, CA, H), f32),
                        pltpu.VMEM((DEPTH, CA, H), f32),
                        pltpu.SemaphoreType.DMA((DEPTH,)),
                        pltpu.SemaphoreType.DMA((DEPTH,))],
        compiler_params=pltpu.CompilerParams(
            dimension_semantics=("parallel", "arbitrary"),
            vmem_limit_bytes=64 * 1024 * 1024),
    )(hidden_flat, *smalls)
    return out.reshape(B, S, H)


# speedup vs baseline: 1.0050x; 1.0050x over previous
import functools

import jax
import jax.numpy as jnp
from jax.experimental import pallas as pl
from jax.experimental.pallas import tpu as pltpu

LN_EPS = 1e-5
V, L, TAU = 32, 8, 1.0


def _ln(x, g, b):
    m = x.mean(-1, keepdims=True)
    v = jnp.mean((x - m) ** 2, axis=-1, keepdims=True)
    return (x - m) * jax.lax.rsqrt(v + LN_EPS) * g + b


def _gelu(x):
    return 0.5 * x * (1.0 + jax.lax.erf(x * 0.7071067811865476))


def _dotT(x, w):
    return jax.lax.dot_general(
        x, w, (((1,), (1,)), ((), ())),
        preferred_element_type=jnp.float32)


def _partial_sum_kernel(h_ref, out_ref):
    j = pl.program_id(1)
    s = jnp.sum(h_ref[...], axis=1)

    @pl.when(j == 0)
    def _():
        out_ref[0] = s

    @pl.when(j != 0)
    def _():
        out_ref[0] += s


def _decode_mlp(parts_ref, gum_ref, w1_ref, b1_ref, g1_ref, be1_ref,
                w2_ref, b2_ref, emb_ref, dw1_ref, db1_ref, g2_ref, be2_ref,
                dw2_ref, db2_ref, n_seq):
    pooled = (parts_ref[0] + parts_ref[1]) * (1.0 / n_seq)
    h = _ln(_dotT(pooled, w1_ref[...]) + b1_ref[...], g1_ref[...], be1_ref[...])
    h = _gelu(h)

    g = -jnp.log(-jnp.log(gum_ref[...]))
    z = _dotT(h, w2_ref[...]) + b2_ref[...] + g

    embed = emb_ref[...]
    acc = jnp.zeros((h.shape[0], dw1_ref.shape[0]), jnp.float32)
    e_dim = emb_ref.shape[1]
    for l in range(L):
        z_l = z[:, l * V:(l + 1) * V]
        m = jnp.max(z_l, axis=-1, keepdims=True)
        iota = jax.lax.broadcasted_iota(jnp.int32, z_l.shape, 1)
        idx = jnp.min(jnp.where(z_l == m, iota, V), axis=-1, keepdims=True)
        y_hard = (iota == idx).astype(jnp.float32)
        emb_l = jnp.dot(y_hard, embed, preferred_element_type=jnp.float32)
        dw1_l = dw1_ref[:, l * e_dim:(l + 1) * e_dim]
        acc = acc + _dotT(emb_l, dw1_l)
    dh = _ln(acc + db1_ref[...], g2_ref[...], be2_ref[...])
    dh = _gelu(dh)
    return _dotT(dh, dw2_ref[...]) + db2_ref[...]


def _add_mlp_kernel(h_ref, parts_ref, gum_ref, w1_ref, b1_ref, g1_ref,
                    be1_ref, w2_ref, b2_ref, emb_ref, dw1_ref, db1_ref,
                    g2_ref, be2_ref, dw2_ref, db2_ref, out_ref, dec_ref,
                    in_ring, out_ring, in_sems, out_sems,
                    *, n_seq, ca, depth, n_blocks, n_chunks):
    i = pl.program_id(0)
    k = pl.program_id(1)
    g = i * n_blocks + k

    def in_cp(blk, slot):
        return pltpu.make_async_copy(
            h_ref.at[pl.ds(blk * ca, ca), :], in_ring.at[slot],
            in_sems.at[slot])

    def out_cp(blk, slot):
        return pltpu.make_async_copy(
            out_ring.at[slot], out_ref.at[pl.ds(blk * ca, ca), :],
            out_sems.at[slot])

    @pl.when(k == 0)
    def _():
        for d in range(depth):
            in_cp(g + d, d).start()
        decoded = _decode_mlp(
            parts_ref, gum_ref, w1_ref, b1_ref, g1_ref, be1_ref, w2_ref,
            b2_ref, emb_ref, dw1_ref, db1_ref, g2_ref, be2_ref, dw2_ref,
            db2_ref, n_seq)
        dec_ref[...] = decoded[:, None, :] * 0.1

    slot = jax.lax.rem(k, depth)
    in_cp(g, slot).wait()

    @pl.when(k >= depth)
    def _():
        out_cp(g - depth, slot).wait()

    b = g // n_chunks
    out_ring[pl.ds(slot, 1)] = (in_ring[pl.ds(slot, 1)]
                                + dec_ref[pl.ds(b, 1), :, :])
    out_cp(g, slot).start()

    @pl.when(k + depth < n_blocks)
    def _():
        in_cp(g + depth, slot).start()

    @pl.when(k == n_blocks - 1)
    def _():
        for d in range(depth):
            blk = n_blocks - depth + d
            out_cp(i * n_blocks + blk, blk % depth).wait()


@jax.jit
def kernel(hidden_states, gumbel_u, enc_w1, enc_b1, ln1_g, ln1_b, enc_w2,
           enc_b2, embed, dec_w1, dec_b1, ln2_g, ln2_b, dec_w2, dec_b2):
    B, S, H = hidden_states.shape
    f32 = jnp.float32
    hidden_flat = hidden_states.reshape(B * S, H)

    CS = 256
    NS = S // CS
    parts = pl.pallas_call(
        _partial_sum_kernel,
        grid=(2, NS // 2),
        in_specs=[pl.BlockSpec((B, CS, H),
                               lambda i, j, nsh=NS // 2: (0, i * nsh + j, 0))],
        out_specs=pl.BlockSpec((1, B, H), lambda i, j: (i, 0, 0)),
        out_shape=jax.ShapeDtypeStruct((2, B, H), f32),
        compiler_params=pltpu.CompilerParams(
            dimension_semantics=("parallel", "arbitrary"),
            vmem_limit_bytes=64 * 1024 * 1024),
    )(hidden_states)

    CA = 512
    DEPTH = 4
    NBLK = (B * S) // CA // 2
    NA = S // CA
    vec = lambda v: v.reshape(1, -1)
    whole = lambda a: pl.BlockSpec(a.shape, lambda i, k: (0,) * a.ndim)
    smalls = (parts, gumbel_u.reshape(B, -1), enc_w1, vec(enc_b1), vec(ln1_g), vec(ln1_b),
              enc_w2, vec(enc_b2), embed, dec_w1, vec(dec_b1), vec(ln2_g),
              vec(ln2_b), dec_w2, vec(dec_b2))
    out = pl.pallas_call(
        functools.partial(_add_mlp_kernel, n_seq=S, ca=CA, depth=DEPTH,
                          n_blocks=NBLK, n_chunks=NA),
        grid=(2, NBLK),
        in_specs=[pl.BlockSpec(memory_space=pl.ANY)]
                 + [whole(a) for a in smalls],
        out_specs=pl.BlockSpec(memory_space=pl.ANY),
        out_shape=jax.ShapeDtypeStruct((B * S, H), f32),
        scratch_shapes=[pltpu.VMEM((B, 1, H), f32),
                        pltpu.VMEM((DEPTH, CA, H), f32),
                        pltpu.VMEM((DEPTH, CA, H), f32),
                        pltpu.SemaphoreType.DMA((DEPTH,)),
                        pltpu.SemaphoreType.DMA((DEPTH,))],
        compiler_params=pltpu.CompilerParams(
            dimension_semantics=("parallel", "arbitrary"),
            vmem_limit_bytes=64 * 1024 * 1024),
    )(hidden_flat, *smalls)
    return out.reshape(B, S, H)


# speedup vs baseline: 1.0052x; 1.0002x over previous
import functools

import jax
import jax.numpy as jnp
from jax.experimental import pallas as pl
from jax.experimental.pallas import tpu as pltpu

LN_EPS = 1e-5
V, L, TAU = 32, 8, 1.0


def _ln(x, g, b):
    m = x.mean(-1, keepdims=True)
    v = jnp.mean((x - m) ** 2, axis=-1, keepdims=True)
    return (x - m) * jax.lax.rsqrt(v + LN_EPS) * g + b


def _gelu(x):
    return 0.5 * x * (1.0 + jax.lax.erf(x * 0.7071067811865476))


def _dotT(x, w):
    return jax.lax.dot_general(
        x, w, (((1,), (1,)), ((), ())),
        preferred_element_type=jnp.float32)


def _partial_sum_kernel(h_ref, out_ref):
    j = pl.program_id(1)
    s = jnp.sum(h_ref[...], axis=1)

    @pl.when(j == 0)
    def _():
        out_ref[0] = s

    @pl.when(j != 0)
    def _():
        out_ref[0] += s


def _decode_mlp(parts_ref, gum_ref, w1_ref, b1_ref, g1_ref, be1_ref,
                w2_ref, b2_ref, emb_ref, dw1_ref, db1_ref, g2_ref, be2_ref,
                dw2_ref, db2_ref, n_seq):
    pooled = (parts_ref[0] + parts_ref[1]) * (1.0 / n_seq)
    h = _ln(_dotT(pooled, w1_ref[...]) + b1_ref[...], g1_ref[...], be1_ref[...])
    h = _gelu(h)

    g = -jnp.log(-jnp.log(gum_ref[...]))
    z = _dotT(h, w2_ref[...]) + b2_ref[...] + g

    embed = emb_ref[...]
    acc = jnp.zeros((h.shape[0], dw1_ref.shape[0]), jnp.float32)
    e_dim = emb_ref.shape[1]
    for l in range(L):
        z_l = z[:, l * V:(l + 1) * V]
        m = jnp.max(z_l, axis=-1, keepdims=True)
        iota = jax.lax.broadcasted_iota(jnp.int32, z_l.shape, 1)
        idx = jnp.min(jnp.where(z_l == m, iota, V), axis=-1, keepdims=True)
        y_hard = (iota == idx).astype(jnp.float32)
        emb_l = jnp.dot(y_hard, embed, preferred_element_type=jnp.float32)
        dw1_l = dw1_ref[:, l * e_dim:(l + 1) * e_dim]
        acc = acc + _dotT(emb_l, dw1_l)
    dh = _ln(acc + db1_ref[...], g2_ref[...], be2_ref[...])
    dh = _gelu(dh)
    return _dotT(dh, dw2_ref[...]) + db2_ref[...]


def _add_mlp_kernel(h_ref, parts_ref, gum_ref, w1_ref, b1_ref, g1_ref,
                    be1_ref, w2_ref, b2_ref, emb_ref, dw1_ref, db1_ref,
                    g2_ref, be2_ref, dw2_ref, db2_ref, out_ref, dec_ref,
                    in_ring, out_ring, in_sems, out_sems,
                    *, n_seq, ca, depth, n_blocks, n_chunks):
    i = pl.program_id(0)
    k = pl.program_id(1)
    g = i * n_blocks + k

    def in_cp(blk, slot):
        return pltpu.make_async_copy(
            h_ref.at[pl.ds(blk * ca, ca), :], in_ring.at[slot],
            in_sems.at[slot])

    def out_cp(blk, slot):
        return pltpu.make_async_copy(
            out_ring.at[slot], out_ref.at[pl.ds(blk * ca, ca), :],
            out_sems.at[slot])

    @pl.when(k == 0)
    def _():
        for d in range(depth):
            in_cp(g + d, d).start()
        decoded = _decode_mlp(
            parts_ref, gum_ref, w1_ref, b1_ref, g1_ref, be1_ref, w2_ref,
            b2_ref, emb_ref, dw1_ref, db1_ref, g2_ref, be2_ref, dw2_ref,
            db2_ref, n_seq)
        dec_ref[...] = decoded[:, None, :] * 0.1

    slot = jax.lax.rem(k, depth)
    in_cp(g, slot).wait()

    @pl.when(k >= depth)
    def _():
        out_cp(g - depth, slot).wait()

    b = g // n_chunks
    out_ring[pl.ds(slot, 1)] = (in_ring[pl.ds(slot, 1)]
                                + dec_ref[pl.ds(b, 1), :, :])
    out_cp(g, slot).start()

    @pl.when(k + depth < n_blocks)
    def _():
        in_cp(g + depth, slot).start()

    @pl.when(k == n_blocks - 1)
    def _():
        for d in range(depth):
            blk = n_blocks - depth + d
            out_cp(i * n_blocks + blk, blk % depth).wait()


@jax.jit
def kernel(hidden_states, gumbel_u, enc_w1, enc_b1, ln1_g, ln1_b, enc_w2,
           enc_b2, embed, dec_w1, dec_b1, ln2_g, ln2_b, dec_w2, dec_b2):
    B, S, H = hidden_states.shape
    f32 = jnp.float32
    hidden_flat = hidden_states.reshape(B * S, H)

    CS = 256
    NS = S // CS
    parts = pl.pallas_call(
        _partial_sum_kernel,
        grid=(2, NS // 2),
        in_specs=[pl.BlockSpec((B, CS, H),
                               lambda i, j, nsh=NS // 2: (0, i * nsh + j, 0))],
        out_specs=pl.BlockSpec((1, B, H), lambda i, j: (i, 0, 0)),
        out_shape=jax.ShapeDtypeStruct((2, B, H), f32),
        compiler_params=pltpu.CompilerParams(
            dimension_semantics=("parallel", "arbitrary"),
            vmem_limit_bytes=64 * 1024 * 1024),
    )(hidden_states)

    CA = 512
    DEPTH = 6
    NBLK = (B * S) // CA // 2
    NA = S // CA
    vec = lambda v: v.reshape(1, -1)
    whole = lambda a: pl.BlockSpec(a.shape, lambda i, k: (0,) * a.ndim)
    smalls = (parts, gumbel_u.reshape(B, -1), enc_w1, vec(enc_b1), vec(ln1_g), vec(ln1_b),
              enc_w2, vec(enc_b2), embed, dec_w1, vec(dec_b1), vec(ln2_g),
              vec(ln2_b), dec_w2, vec(dec_b2))
    out = pl.pallas_call(
        functools.partial(_add_mlp_kernel, n_seq=S, ca=CA, depth=DEPTH,
                          n_blocks=NBLK, n_chunks=NA),
        grid=(2, NBLK),
        in_specs=[pl.BlockSpec(memory_space=pl.ANY)]
                 + [whole(a) for a in smalls],
        out_specs=pl.BlockSpec(memory_space=pl.ANY),
        out_shape=jax.ShapeDtypeStruct((B * S, H), f32),
        scratch_shapes=[pltpu.VMEM((B, 1, H), f32),
                        pltpu.VMEM((DEPTH, CA, H), f32),
                        pltpu.VMEM((DEPTH, CA, H), f32),
                        pltpu.SemaphoreType.DMA((DEPTH,)),
                        pltpu.SemaphoreType.DMA((DEPTH,))],
        compiler_params=pltpu.CompilerParams(
            dimension_semantics=("parallel", "arbitrary"),
            vmem_limit_bytes=64 * 1024 * 1024),
    )(hidden_flat, *smalls)
    return out.reshape(B, S, H)
